# Initial kernel scaffold; baseline (speedup 1.0000x reference)
import jax
import jax.numpy as jnp
from jax.experimental import pallas as pl


def kernel(x, expert_w, expert_b, gate_w, gate_b):
    raise NotImplementedError("write your pallas kernel here")


# speedup vs baseline: 6.4510x; 6.4510x over previous
import functools

import jax
import jax.numpy as jnp
from jax.experimental import pallas as pl


def _router_pool_body(x_ref, gw_ref, gb_ref, eb_ref, v_ref, t0_ref, s_ref):
    i = pl.program_id(0)
    n = pl.num_programs(0)
    xb = x_ref[...]
    logits = jax.lax.dot_general(
        xb, gw_ref[...], (((1,), (1,)), ((), ())),
        preferred_element_type=jnp.float32) + gb_ref[...]
    m = jnp.max(logits, axis=-1, keepdims=True)
    e = jnp.exp(logits - m)
    w = e / jnp.sum(e, axis=-1, keepdims=True)

    @pl.when(i == 0)
    def _():
        v_ref[...] = jnp.zeros_like(v_ref)
        s_ref[...] = jnp.zeros_like(s_ref)

    v_ref[...] += jax.lax.dot_general(
        w, xb, (((0,), (0,)), ((), ())),
        preferred_element_type=jnp.float32)
    s_ref[...] += jnp.sum(w, axis=0, keepdims=True)

    @pl.when(i == n - 1)
    def _():
        t0_ref[...] = jax.lax.dot_general(
            s_ref[...], eb_ref[...], (((1,), (0,)), ((), ())),
            preferred_element_type=jnp.float32)


def _contract_body(v_ref, ew_ref, t0_ref, t_ref):
    e = pl.program_id(0)

    @pl.when(e == 0)
    def _():
        t_ref[...] = t0_ref[...]

    t_ref[...] += jax.lax.dot_general(
        v_ref[pl.ds(e, 1), :], ew_ref[0], (((1,), (1,)), ((), ())),
        preferred_element_type=jnp.float32)


def _broadcast_body(t_ref, out_ref):
    out_ref[...] = jnp.broadcast_to(t_ref[...], out_ref.shape)


@jax.jit
def kernel(x, expert_w, expert_b, gate_w, gate_b):
    B, I = x.shape
    E, O, _ = expert_w.shape
    TB = 512
    nblk = B // TB

    v, t0, _s = pl.pallas_call(
        _router_pool_body,
        grid=(nblk,),
        in_specs=[
            pl.BlockSpec((TB, I), lambda i: (i, 0)),
            pl.BlockSpec((E, I), lambda i: (0, 0)),
            pl.BlockSpec((1, E), lambda i: (0, 0)),
            pl.BlockSpec((E, O), lambda i: (0, 0)),
        ],
        out_specs=[
            pl.BlockSpec((E, I), lambda i: (0, 0)),
            pl.BlockSpec((1, O), lambda i: (0, 0)),
            pl.BlockSpec((1, E), lambda i: (0, 0)),
        ],
        out_shape=[
            jax.ShapeDtypeStruct((E, I), jnp.float32),
            jax.ShapeDtypeStruct((1, O), jnp.float32),
            jax.ShapeDtypeStruct((1, E), jnp.float32),
        ],
    )(x, gate_w, gate_b.reshape(1, E), expert_b)

    total = pl.pallas_call(
        _contract_body,
        grid=(E,),
        in_specs=[
            pl.BlockSpec((E, I), lambda e: (0, 0)),
            pl.BlockSpec((1, O, I), lambda e: (e, 0, 0)),
            pl.BlockSpec((1, O), lambda e: (0, 0)),
        ],
        out_specs=pl.BlockSpec((1, O), lambda e: (0, 0)),
        out_shape=jax.ShapeDtypeStruct((1, O), jnp.float32),
    )(v, expert_w, t0)

    return pl.pallas_call(
        _broadcast_body,
        grid=(nblk,),
        in_specs=[pl.BlockSpec((1, O), lambda i: (0, 0))],
        out_specs=pl.BlockSpec((TB, O), lambda i: (i, 0)),
        out_shape=jax.ShapeDtypeStruct((B, O), jnp.float32),
    )(total)


# speedup vs baseline: 6.8195x; 1.0571x over previous
import jax
import jax.numpy as jnp
from jax.experimental import pallas as pl
from jax.experimental.pallas import tpu as pltpu


def _fused_body(x_ref, gw_ref, gb_ref, eb_ref, ew_ref, out_ref,
                v_ref, s_ref, t_ref, *, nb, ne, nob):
    i = pl.program_id(0)

    @pl.when(i < nb)
    def _router():
        xb = x_ref[...]
        logits = jax.lax.dot_general(
            xb, gw_ref[...], (((1,), (1,)), ((), ())),
            preferred_element_type=jnp.float32) + gb_ref[...]
        m = jnp.max(logits, axis=-1, keepdims=True)
        ex = jnp.exp(logits - m)
        w = ex / jnp.sum(ex, axis=-1, keepdims=True)

        @pl.when(i == 0)
        def _():
            v_ref[...] = jnp.zeros_like(v_ref)
            s_ref[...] = jnp.zeros_like(s_ref)

        v_ref[...] += jax.lax.dot_general(
            w, xb, (((0,), (0,)), ((), ())),
            preferred_element_type=jnp.float32)
        s_ref[...] += jnp.sum(w, axis=0, keepdims=True)

    @pl.when((i >= nb) & (i < nb + ne))
    def _contract():
        e = i - nb

        @pl.when(e == 0)
        def _():
            t_ref[...] = jax.lax.dot_general(
                s_ref[...], eb_ref[...], (((1,), (0,)), ((), ())),
                preferred_element_type=jnp.float32)

        t_ref[...] += jax.lax.dot_general(
            v_ref[pl.ds(e, 1), :], ew_ref[0], (((1,), (1,)), ((), ())),
            preferred_element_type=jnp.float32)

    @pl.when(i >= nb + ne)
    def _broadcast():
        out_ref[...] = jnp.broadcast_to(t_ref[...], out_ref.shape)


@jax.jit
def kernel(x, expert_w, expert_b, gate_w, gate_b):
    B, I = x.shape
    E, O, _ = expert_w.shape
    TB = 512
    OB = 1024
    nb, ne, nob = B // TB, E, B // OB
    nsteps = nb + ne + nob

    import functools
    body = functools.partial(_fused_body, nb=nb, ne=ne, nob=nob)

    return pl.pallas_call(
        body,
        grid=(nsteps,),
        in_specs=[
            pl.BlockSpec((TB, I), lambda i: (jnp.minimum(i, nb - 1), 0)),
            pl.BlockSpec((E, I), lambda i: (0, 0)),
            pl.BlockSpec((1, E), lambda i: (0, 0)),
            pl.BlockSpec((E, O), lambda i: (0, 0)),
            pl.BlockSpec((1, O, I),
                         lambda i: (jnp.clip(i - nb, 0, ne - 1), 0, 0)),
        ],
        out_specs=pl.BlockSpec((OB, O),
                               lambda i: (jnp.clip(i - (nb + ne), 0, nob - 1), 0)),
        out_shape=jax.ShapeDtypeStruct((B, O), jnp.float32),
        scratch_shapes=[
            pltpu.VMEM((E, I), jnp.float32),
            pltpu.VMEM((1, E), jnp.float32),
            pltpu.VMEM((1, O), jnp.float32),
        ],
    )(x, gate_w, gate_b.reshape(1, E), expert_b, expert_w)
